# baseline (speedup 1.0000x reference)
import functools

import jax
import jax.numpy as jnp
from jax import lax
from jax.experimental import pallas as pl
from jax.experimental.pallas import tpu as pltpu
from jax.experimental.pallas import tpu_sc as plsc

N_COLS = 26
VOCAB = 100000
B = 16384
SUB = 32
HID = 256
ENT = 16

GW = 128
DW = N_COLS * GW

NC = 2
NS = 16
NW = NC * NS

RB = 512
SUBCH = RB // 128
RBLOCKS = B // RB
TOTAL_CHUNKS = N_COLS * RBLOCKS
CHUNKS_PER_W = TOTAL_CHUNKS // NW


def _sc_gather(tg, idx3d):
    mesh = plsc.VectorSubcoreMesh(core_axis_name="c", subcore_axis_name="s")

    @functools.partial(
        pl.kernel,
        out_type=jax.ShapeDtypeStruct((B, DW), jnp.float32),
        mesh=mesh,
        scratch_types=[
            pltpu.VMEM((SUBCH, 128), jnp.int32),
            pltpu.VMEM((RB, GW), jnp.float32),
            pltpu.SemaphoreType.DMA,
        ],
    )
    def gather_kernel(tg_hbm, idx_hbm, out_hbm, idx_v, rows_v, sem):
        wid = lax.axis_index("s") * NC + lax.axis_index("c")

        def body(i, carry):
            k = wid * CHUNKS_PER_W + i
            c = k // RBLOCKS
            rb = k % RBLOCKS
            pltpu.sync_copy(idx_hbm.at[k], idx_v)
            copies = [
                pltpu.async_copy(
                    tg_hbm.at[idx_v.at[j]],
                    rows_v.at[pl.ds(j * 128, 128), :],
                    sem,
                )
                for j in range(SUBCH)
            ]
            for cp in copies:
                cp.wait()
            pltpu.sync_copy(
                rows_v,
                out_hbm.at[pl.ds(rb * RB, RB), pl.ds(c * GW, GW)],
            )
            return carry

        lax.fori_loop(0, CHUNKS_PER_W, body, 0)

    return gather_kernel(tg, idx3d)


def _tc_mlp(wide, qarr, W1rep, b1, W2, b2, W3, b3):
    BLK = 1024

    def body(wide_ref, q_ref, w1_ref, b1_ref, w2_ref, b2_ref, w3_ref, b3_ref,
             out_ref):
        t = lax.broadcasted_iota(jnp.int32, (1, DW), 1)
        tq = (t % GW) // SUB
        qe = jnp.broadcast_to(
            q_ref[...][:, :, None], (BLK, N_COLS, GW)).reshape(BLK, DW)
        w = jnp.where(qe == tq, wide_ref[...], 0.0).astype(jnp.bfloat16)
        h = jnp.dot(w, w1_ref[...], preferred_element_type=jnp.float32)
        h = jnp.maximum(h + b1_ref[...], 0.0)
        h = jnp.dot(h, w2_ref[...], preferred_element_type=jnp.float32)
        h = jnp.maximum(h + b2_ref[...], 0.0)
        out_ref[...] = (
            jnp.dot(h, w3_ref[...], preferred_element_type=jnp.float32) + b3_ref[...]
        )

    return pl.pallas_call(
        body,
        grid=(B // BLK,),
        in_specs=[
            pl.BlockSpec((BLK, DW), lambda i: (i, 0)),
            pl.BlockSpec((BLK, N_COLS), lambda i: (i, 0)),
            pl.BlockSpec((DW, HID), lambda i: (0, 0)),
            pl.BlockSpec((1, HID), lambda i: (0, 0)),
            pl.BlockSpec((HID, ENT), lambda i: (0, 0)),
            pl.BlockSpec((1, ENT), lambda i: (0, 0)),
            pl.BlockSpec((ENT, ENT), lambda i: (0, 0)),
            pl.BlockSpec((1, ENT), lambda i: (0, 0)),
        ],
        out_specs=pl.BlockSpec((BLK, ENT), lambda i: (i, 0)),
        out_shape=jax.ShapeDtypeStruct((B, ENT), jnp.float32),
    )(wide, qarr, W1rep, b1.reshape(1, HID), W2, b2.reshape(1, ENT), W3,
      b3.reshape(1, ENT))


def kernel(col_0, col_1, col_2, col_3, col_4, col_5, col_6, col_7, col_8,
           col_9, col_10, col_11, col_12, col_13, col_14, col_15, col_16,
           col_17, col_18, col_19, col_20, col_21, col_22, col_23, col_24,
           col_25, tables, W1, b1, W2, b2, W3, b3):
    cols = jnp.stack([col_0, col_1, col_2, col_3, col_4, col_5, col_6, col_7,
                      col_8, col_9, col_10, col_11, col_12, col_13, col_14,
                      col_15, col_16, col_17, col_18, col_19, col_20, col_21,
                      col_22, col_23, col_24, col_25]).astype(jnp.int32)
    offs = (jnp.arange(N_COLS, dtype=jnp.int32) * (VOCAB // 4))[:, None]
    idx3d = ((cols // 4) + offs).reshape(TOTAL_CHUNKS, SUBCH, 128)
    qarr = (cols % 4).T
    tg = jnp.concatenate(
        [tables[:, q::4, :] for q in range(4)], axis=2
    ).reshape(N_COLS * VOCAB // 4, GW)
    W1rep = jnp.broadcast_to(
        W1.reshape(N_COLS, 1, SUB, HID), (N_COLS, 4, SUB, HID)
    ).reshape(DW, HID).astype(jnp.bfloat16)
    wide = _sc_gather(tg, idx3d)
    return _tc_mlp(wide, qarr, W1rep, b1, W2, b2, W3, b3)


# speedup vs baseline: 8.7021x; 8.7021x over previous
import functools

import jax
import jax.numpy as jnp
from jax import lax
from jax.experimental import pallas as pl
from jax.experimental.pallas import tpu as pltpu
from jax.experimental.pallas import tpu_sc as plsc

N_COLS = 26
VOCAB = 100000
B = 16384
SUB = 32
HID = 256
ENT = 16

GW = 128
DW = N_COLS * GW

NC = 2
NS = 16
NW = NC * NS

RB = 512
SUBCH = RB // 128
RBLOCKS = B // RB
TOTAL_CHUNKS = N_COLS * RBLOCKS
CHUNKS_PER_W = TOTAL_CHUNKS // NW


def _sc_gather(tflat, idx3d):
    mesh = plsc.VectorSubcoreMesh(core_axis_name="c", subcore_axis_name="s")

    @functools.partial(
        pl.kernel,
        out_type=jax.ShapeDtypeStruct((B, DW), jnp.float32),
        mesh=mesh,
        scratch_types=[
            pltpu.VMEM((SUBCH, 128), jnp.int32),
            [pltpu.VMEM((128, GW), jnp.float32) for _ in range(SUBCH)],
            pltpu.SemaphoreType.DMA,
        ],
    )
    def gather_kernel(t_hbm, idx_hbm, out_hbm, idx_v, bufs, sem):
        wid = lax.axis_index("s") * NC + lax.axis_index("c")

        def body(i, carry):
            k = wid * CHUNKS_PER_W + i
            c = k // RBLOCKS
            rb = k % RBLOCKS
            pltpu.sync_copy(idx_hbm.at[k], idx_v)
            copies = [
                pltpu.async_copy(t_hbm.at[idx_v.at[j]], bufs[j], sem)
                for j in range(SUBCH)
            ]
            for j, cp in enumerate(copies):
                cp.wait()
                pltpu.sync_copy(
                    bufs[j],
                    out_hbm.at[pl.ds(rb * RB + j * 128, 128),
                               pl.ds(c * GW, GW)],
                )
            return carry

        lax.fori_loop(0, CHUNKS_PER_W, body, 0)

    return gather_kernel(tflat, idx3d)


def _tc_mlp(wide, W1pad, b1, W2, b2, W3, b3):
    BLK = 1024

    def body(wide_ref, w1_ref, b1_ref, w2_ref, b2_ref, w3_ref, b3_ref,
             out_ref):
        w = wide_ref[...].astype(jnp.bfloat16)
        h = jnp.dot(w, w1_ref[...], preferred_element_type=jnp.float32)
        h = jnp.maximum(h + b1_ref[...], 0.0)
        h = jnp.dot(h, w2_ref[...], preferred_element_type=jnp.float32)
        h = jnp.maximum(h + b2_ref[...], 0.0)
        out_ref[...] = (
            jnp.dot(h, w3_ref[...], preferred_element_type=jnp.float32) + b3_ref[...]
        )

    return pl.pallas_call(
        body,
        grid=(B // BLK,),
        in_specs=[
            pl.BlockSpec((BLK, DW), lambda i: (i, 0)),
            pl.BlockSpec((DW, HID), lambda i: (0, 0)),
            pl.BlockSpec((1, HID), lambda i: (0, 0)),
            pl.BlockSpec((HID, ENT), lambda i: (0, 0)),
            pl.BlockSpec((1, ENT), lambda i: (0, 0)),
            pl.BlockSpec((ENT, ENT), lambda i: (0, 0)),
            pl.BlockSpec((1, ENT), lambda i: (0, 0)),
        ],
        out_specs=pl.BlockSpec((BLK, ENT), lambda i: (i, 0)),
        out_shape=jax.ShapeDtypeStruct((B, ENT), jnp.float32),
    )(wide, W1pad, b1.reshape(1, HID), W2, b2.reshape(1, ENT), W3,
      b3.reshape(1, ENT))


def kernel(col_0, col_1, col_2, col_3, col_4, col_5, col_6, col_7, col_8,
           col_9, col_10, col_11, col_12, col_13, col_14, col_15, col_16,
           col_17, col_18, col_19, col_20, col_21, col_22, col_23, col_24,
           col_25, tables, W1, b1, W2, b2, W3, b3):
    cols = jnp.stack([col_0, col_1, col_2, col_3, col_4, col_5, col_6, col_7,
                      col_8, col_9, col_10, col_11, col_12, col_13, col_14,
                      col_15, col_16, col_17, col_18, col_19, col_20, col_21,
                      col_22, col_23, col_24, col_25]).astype(jnp.int32)
    offs = (jnp.arange(N_COLS, dtype=jnp.int32) * VOCAB)[:, None]
    idx3d = (cols + offs).reshape(TOTAL_CHUNKS, SUBCH, 128)
    tflat = jnp.pad(tables, ((0, 0), (0, 0), (0, GW - SUB))).reshape(
        N_COLS * VOCAB, GW)
    W1pad = jnp.pad(
        W1.reshape(N_COLS, SUB, HID), ((0, 0), (0, GW - SUB), (0, 0))
    ).reshape(DW, HID).astype(jnp.bfloat16)
    wide = _sc_gather(tflat, idx3d)
    return _tc_mlp(wide, W1pad, b1, W2, b2, W3, b3)
